# baseline (speedup 1.0000x reference)
import functools

import jax
import jax.numpy as jnp
from jax import lax
from jax.experimental import pallas as pl
from jax.experimental.pallas import tpu as pltpu
from jax.experimental.pallas import tpu_sc as plsc

_K = 128
_D = 32
_L = 8
_B = 16384
_BB = 256
_BF = _BB * _L

_NW = 32
_RPW = _B // _NW
_KC = 8
_SC_MESH = plsc.VectorSubcoreMesh(core_axis_name="c", subcore_axis_name="s")


@functools.partial(
    pl.kernel,
    mesh=_SC_MESH,
    out_type=jax.ShapeDtypeStruct((_B, _K, _D), jnp.float32),
    scratch_types=[
        pltpu.VMEM((_KC, _K, _D), jnp.float32),
        pltpu.SemaphoreType.DMA,
    ],
)
def _bcast_sc(cb_hbm, out_hbm, buf, sem):
    wid = lax.axis_index("s") * 2 + lax.axis_index("c")
    base = wid * _RPW
    for i in range(_KC):
        pltpu.sync_copy(cb_hbm, buf.at[i])

    def body(t, carry):
        pltpu.sync_copy(buf, out_hbm.at[pl.ds(base + t * _KC, _KC)])
        return carry

    lax.fori_loop(0, _RPW // _KC, body, 0)


def _tc_body(x_ref, cb_ref, q_ref, pol_ref):
    x = x_ref[...]
    cb = cb_ref[...]
    c2 = jnp.sum(cb ** 2, axis=-1)
    x2 = jnp.sum(x ** 2, axis=-1, keepdims=True)
    m = lax.dot_general(x, cb, (((1,), (1,)), ((), ())),
                        preferred_element_type=jnp.float32)
    d = (x2 + c2[None, :]) - 2.0 * m
    dmin = jnp.min(d, axis=1, keepdims=True)
    iota = lax.broadcasted_iota(jnp.int32, (_BF, _K), 1)
    idx = jnp.min(jnp.where(d <= dmin, iota, _K), axis=1, keepdims=True)
    onehot = (iota == idx).astype(jnp.float32)
    q = lax.dot_general(onehot, cb, (((1,), (0,)), ((), ())),
                        preferred_element_type=jnp.float32)
    q_ref[...] = q
    pol_ref[...] = x + (q - x)


@jax.jit
def kernel(latent, codebook):
    xflat = latent.reshape(_B * _L, _D)
    grid = _B // _BB
    q, pol = pl.pallas_call(
        _tc_body,
        grid=(grid,),
        in_specs=[
            pl.BlockSpec((_BF, _D), lambda i: (i, 0)),
            pl.BlockSpec((_K, _D), lambda i: (0, 0)),
        ],
        out_specs=[
            pl.BlockSpec((_BF, _D), lambda i: (i, 0)),
            pl.BlockSpec((_BF, _D), lambda i: (i, 0)),
        ],
        out_shape=[
            jax.ShapeDtypeStruct((_B * _L, _D), jnp.float32),
            jax.ShapeDtypeStruct((_B * _L, _D), jnp.float32),
        ],
    )(xflat, codebook)
    cbset = _bcast_sc(codebook)
    return (pol.reshape(_B, _L * _D),
            q.reshape(_B, _L * _D),
            cbset)


# speedup vs baseline: 3.8442x; 3.8442x over previous
import functools

import jax
import jax.numpy as jnp
from jax import lax
from jax.experimental import pallas as pl

_K = 128
_D = 32
_L = 8
_B = 16384
_BB = 256
_BF = _BB * _L


def _body(x_ref, cb_ref, cbt_ref, q_ref, pol_ref, set_ref):
    x = x_ref[...]
    cb = cb_ref[...]
    cbt = cbt_ref[...]
    c2 = jnp.sum(cb ** 2, axis=-1)
    x2 = jnp.sum(x ** 2, axis=-1, keepdims=True)
    m = lax.dot_general(x, cb, (((1,), (1,)), ((), ())),
                        preferred_element_type=jnp.float32)
    d = (x2 + c2[None, :]) - 2.0 * m
    dmin = jnp.min(d, axis=1, keepdims=True)
    iota = lax.broadcasted_iota(jnp.int32, (_BF, _K), 1)
    idx = jnp.min(jnp.where(d <= dmin, iota, _K), axis=1, keepdims=True)
    onehot = (iota == idx).astype(jnp.float32)
    q = lax.dot_general(onehot, cb, (((1,), (0,)), ((), ())),
                        preferred_element_type=jnp.float32)
    q_ref[...] = q
    pol_ref[...] = x + (q - x)
    set_ref[...] = jnp.broadcast_to(cbt[None], (_BB, _D, _K))


@jax.jit
def kernel(latent, codebook):
    xflat = latent.reshape(_B * _L, _D)
    cbt = codebook.T
    grid = _B // _BB
    q, pol, cbset_t = pl.pallas_call(
        _body,
        grid=(grid,),
        in_specs=[
            pl.BlockSpec((_BF, _D), lambda i: (i, 0)),
            pl.BlockSpec((_K, _D), lambda i: (0, 0)),
            pl.BlockSpec((_D, _K), lambda i: (0, 0)),
        ],
        out_specs=[
            pl.BlockSpec((_BF, _D), lambda i: (i, 0)),
            pl.BlockSpec((_BF, _D), lambda i: (i, 0)),
            pl.BlockSpec((_BB, _D, _K), lambda i: (i, 0, 0)),
        ],
        out_shape=[
            jax.ShapeDtypeStruct((_B * _L, _D), jnp.float32),
            jax.ShapeDtypeStruct((_B * _L, _D), jnp.float32),
            jax.ShapeDtypeStruct((_B, _D, _K), jnp.float32),
        ],
    )(xflat, codebook, cbt)
    return (pol.reshape(_B, _L * _D),
            q.reshape(_B, _L * _D),
            jnp.transpose(cbset_t, (0, 2, 1)))


# speedup vs baseline: 4.9898x; 1.2980x over previous
import functools

import jax
import jax.numpy as jnp
from jax import lax
from jax.experimental import pallas as pl

_K = 128
_D = 32
_L = 8
_B = 16384
_BB = 256
_W = _L * _K


def _body(x_ref, x2_ref, c2_ref, w1_ref, w2_ref, e_ref, cbt_ref,
          q_ref, pol_ref, set_ref):
    x = x_ref[...]
    m = lax.dot_general(x, w1_ref[...], (((1,), (0,)), ((), ())),
                        preferred_element_type=jnp.float32)
    x2seg = lax.dot_general(x2_ref[...], e_ref[...], (((1,), (0,)), ((), ())),
                            preferred_element_type=jnp.float32)
    d = (x2seg + c2_ref[...]) - 2.0 * m
    iota = lax.broadcasted_iota(jnp.int32, (_BB, _K), 1)
    parts = []
    for j in range(_L):
        dj = d[:, j * _K:(j + 1) * _K]
        dmin = jnp.min(dj, axis=1, keepdims=True)
        idx = jnp.min(jnp.where(dj <= dmin, iota, _K), axis=1, keepdims=True)
        parts.append((iota == idx).astype(jnp.float32))
    onehot = jnp.concatenate(parts, axis=1)
    q = lax.dot_general(onehot, w2_ref[...], (((1,), (0,)), ((), ())),
                        preferred_element_type=jnp.float32)
    q_ref[...] = q
    pol_ref[...] = x + (q - x)
    set_ref[...] = jnp.broadcast_to(cbt_ref[...][None], (_BB, _D, _K))


@jax.jit
def kernel(latent, codebook):
    cbt = codebook.T
    w1 = jax.scipy.linalg.block_diag(*([cbt] * _L))
    w2 = jax.scipy.linalg.block_diag(*([codebook] * _L))
    e = jax.scipy.linalg.block_diag(*([jnp.ones((1, _K), jnp.float32)] * _L))
    c2 = jnp.sum(codebook ** 2, axis=-1)
    c2seg = jnp.tile(c2, _L)[None, :]
    x2 = jnp.sum(latent.reshape(_B * _L, _D) ** 2, axis=-1).reshape(_B, _L)
    grid = _B // _BB
    q, pol, cbset_t = pl.pallas_call(
        _body,
        grid=(grid,),
        in_specs=[
            pl.BlockSpec((_BB, _L * _D), lambda i: (i, 0)),
            pl.BlockSpec((_BB, _L), lambda i: (i, 0)),
            pl.BlockSpec((1, _W), lambda i: (0, 0)),
            pl.BlockSpec((_L * _D, _W), lambda i: (0, 0)),
            pl.BlockSpec((_W, _L * _D), lambda i: (0, 0)),
            pl.BlockSpec((_L, _W), lambda i: (0, 0)),
            pl.BlockSpec((_D, _K), lambda i: (0, 0)),
        ],
        out_specs=[
            pl.BlockSpec((_BB, _L * _D), lambda i: (i, 0)),
            pl.BlockSpec((_BB, _L * _D), lambda i: (i, 0)),
            pl.BlockSpec((_BB, _D, _K), lambda i: (i, 0, 0)),
        ],
        out_shape=[
            jax.ShapeDtypeStruct((_B, _L * _D), jnp.float32),
            jax.ShapeDtypeStruct((_B, _L * _D), jnp.float32),
            jax.ShapeDtypeStruct((_B, _D, _K), jnp.float32),
        ],
    )(latent, x2, c2seg, w1, w2, e, cbt)
    return (pol, q, jnp.transpose(cbset_t, (0, 2, 1)))


# speedup vs baseline: 5.2732x; 1.0568x over previous
import functools

import jax
import jax.numpy as jnp
from jax import lax
from jax.experimental import pallas as pl

_K = 128
_D = 32
_L = 8
_B = 16384
_BB = 512
_W = _L * _K


def _body(x_ref, x2_ref, c2_ref, w1_ref, w2_ref, e_ref, cbt_ref,
          q_ref, pol_ref, set_ref):
    x = x_ref[...]
    m = lax.dot_general(x, w1_ref[...], (((1,), (0,)), ((), ())),
                        preferred_element_type=jnp.float32)
    x2seg = lax.dot_general(x2_ref[...], e_ref[...], (((1,), (0,)), ((), ())),
                            preferred_element_type=jnp.float32)
    d = (x2seg + c2_ref[...]) - 2.0 * m
    iota = lax.broadcasted_iota(jnp.int32, (_BB, _K), 1)
    parts = []
    for j in range(_L):
        dj = d[:, j * _K:(j + 1) * _K]
        dmin = jnp.min(dj, axis=1, keepdims=True)
        idx = jnp.min(jnp.where(dj <= dmin, iota, _K), axis=1, keepdims=True)
        parts.append((iota == idx).astype(jnp.float32))
    onehot = jnp.concatenate(parts, axis=1)
    q = lax.dot_general(onehot, w2_ref[...], (((1,), (0,)), ((), ())),
                        preferred_element_type=jnp.float32)
    q_ref[...] = q
    pol_ref[...] = x + (q - x)
    set_ref[...] = jnp.broadcast_to(cbt_ref[...][None], (_BB, _D, _K))


@jax.jit
def kernel(latent, codebook):
    cbt = codebook.T
    w1 = jax.scipy.linalg.block_diag(*([cbt] * _L))
    w2 = jax.scipy.linalg.block_diag(*([codebook] * _L))
    e = jax.scipy.linalg.block_diag(*([jnp.ones((1, _K), jnp.float32)] * _L))
    c2 = jnp.sum(codebook ** 2, axis=-1)
    c2seg = jnp.tile(c2, _L)[None, :]
    x2 = jnp.sum(latent.reshape(_B * _L, _D) ** 2, axis=-1).reshape(_B, _L)
    grid = _B // _BB
    q, pol, cbset_t = pl.pallas_call(
        _body,
        grid=(grid,),
        in_specs=[
            pl.BlockSpec((_BB, _L * _D), lambda i: (i, 0)),
            pl.BlockSpec((_BB, _L), lambda i: (i, 0)),
            pl.BlockSpec((1, _W), lambda i: (0, 0)),
            pl.BlockSpec((_L * _D, _W), lambda i: (0, 0)),
            pl.BlockSpec((_W, _L * _D), lambda i: (0, 0)),
            pl.BlockSpec((_L, _W), lambda i: (0, 0)),
            pl.BlockSpec((_D, _K), lambda i: (0, 0)),
        ],
        out_specs=[
            pl.BlockSpec((_BB, _L * _D), lambda i: (i, 0)),
            pl.BlockSpec((_BB, _L * _D), lambda i: (i, 0)),
            pl.BlockSpec((_BB, _D, _K), lambda i: (i, 0, 0)),
        ],
        out_shape=[
            jax.ShapeDtypeStruct((_B, _L * _D), jnp.float32),
            jax.ShapeDtypeStruct((_B, _L * _D), jnp.float32),
            jax.ShapeDtypeStruct((_B, _D, _K), jnp.float32),
        ],
    )(latent, x2, c2seg, w1, w2, e, cbt)
    return (pol, q, jnp.transpose(cbset_t, (0, 2, 1)))


# speedup vs baseline: 5.3179x; 1.0085x over previous
import functools

import jax
import jax.numpy as jnp
from jax import lax
from jax.experimental import pallas as pl

_K = 128
_D = 32
_L = 8
_B = 16384
_BB = 1024
_W = _L * _K


def _body(x_ref, x2_ref, c2_ref, w1_ref, w2_ref, e_ref, cbt_ref,
          q_ref, pol_ref, set_ref):
    x = x_ref[...]
    m = lax.dot_general(x, w1_ref[...], (((1,), (0,)), ((), ())),
                        preferred_element_type=jnp.float32)
    x2seg = lax.dot_general(x2_ref[...], e_ref[...], (((1,), (0,)), ((), ())),
                            preferred_element_type=jnp.float32)
    d = (x2seg + c2_ref[...]) - 2.0 * m
    iota = lax.broadcasted_iota(jnp.int32, (_BB, _K), 1)
    parts = []
    for j in range(_L):
        dj = d[:, j * _K:(j + 1) * _K]
        dmin = jnp.min(dj, axis=1, keepdims=True)
        idx = jnp.min(jnp.where(dj <= dmin, iota, _K), axis=1, keepdims=True)
        parts.append((iota == idx).astype(jnp.float32))
    onehot = jnp.concatenate(parts, axis=1)
    q = lax.dot_general(onehot, w2_ref[...], (((1,), (0,)), ((), ())),
                        preferred_element_type=jnp.float32)
    q_ref[...] = q
    pol_ref[...] = x + (q - x)
    set_ref[...] = jnp.broadcast_to(cbt_ref[...][None], (_BB, _D, _K))


@jax.jit
def kernel(latent, codebook):
    cbt = codebook.T
    w1 = jax.scipy.linalg.block_diag(*([cbt] * _L))
    w2 = jax.scipy.linalg.block_diag(*([codebook] * _L))
    e = jax.scipy.linalg.block_diag(*([jnp.ones((1, _K), jnp.float32)] * _L))
    c2 = jnp.sum(codebook ** 2, axis=-1)
    c2seg = jnp.tile(c2, _L)[None, :]
    x2 = jnp.sum(latent.reshape(_B * _L, _D) ** 2, axis=-1).reshape(_B, _L)
    grid = _B // _BB
    q, pol, cbset_t = pl.pallas_call(
        _body,
        grid=(grid,),
        in_specs=[
            pl.BlockSpec((_BB, _L * _D), lambda i: (i, 0)),
            pl.BlockSpec((_BB, _L), lambda i: (i, 0)),
            pl.BlockSpec((1, _W), lambda i: (0, 0)),
            pl.BlockSpec((_L * _D, _W), lambda i: (0, 0)),
            pl.BlockSpec((_W, _L * _D), lambda i: (0, 0)),
            pl.BlockSpec((_L, _W), lambda i: (0, 0)),
            pl.BlockSpec((_D, _K), lambda i: (0, 0)),
        ],
        out_specs=[
            pl.BlockSpec((_BB, _L * _D), lambda i: (i, 0)),
            pl.BlockSpec((_BB, _L * _D), lambda i: (i, 0)),
            pl.BlockSpec((_BB, _D, _K), lambda i: (i, 0, 0)),
        ],
        out_shape=[
            jax.ShapeDtypeStruct((_B, _L * _D), jnp.float32),
            jax.ShapeDtypeStruct((_B, _L * _D), jnp.float32),
            jax.ShapeDtypeStruct((_B, _D, _K), jnp.float32),
        ],
    )(latent, x2, c2seg, w1, w2, e, cbt)
    return (pol, q, jnp.transpose(cbset_t, (0, 2, 1)))


# speedup vs baseline: 5.3203x; 1.0004x over previous
import jax
import jax.numpy as jnp
from jax import lax
from jax.experimental import pallas as pl

_K = 128
_D = 32
_L = 8
_B = 16384
_BB = 1024
_W = _L * _K


def _body(x_ref, x2_ref, c2_ref, w1_ref, w2_ref, e_ref, cbt_ref,
          q_ref, pol_ref, set_ref):
    x = x_ref[...]
    m = lax.dot_general(x, w1_ref[...], (((1,), (0,)), ((), ())),
                        preferred_element_type=jnp.float32)
    x2seg = lax.dot_general(x2_ref[...], e_ref[...], (((1,), (0,)), ((), ())),
                            preferred_element_type=jnp.float32)
    d = (x2seg + c2_ref[...]) - 2.0 * m
    iota = lax.broadcasted_iota(jnp.int32, (_BB, _K), 1)
    parts = []
    for j in range(_L):
        dj = d[:, j * _K:(j + 1) * _K]
        dmin = jnp.min(dj, axis=1, keepdims=True)
        idx = jnp.min(jnp.where(dj <= dmin, iota, _K), axis=1, keepdims=True)
        parts.append((iota == idx).astype(jnp.float32))
    onehot = jnp.concatenate(parts, axis=1)
    q = lax.dot_general(onehot, w2_ref[...], (((1,), (0,)), ((), ())),
                        preferred_element_type=jnp.float32)
    q_ref[...] = q
    pol_ref[...] = x + (q - x)
    set_ref[...] = jnp.broadcast_to(cbt_ref[...][None], (_BB, _D, _K))


@jax.jit
def kernel(latent, codebook):
    cbt = codebook.T
    w1 = jax.scipy.linalg.block_diag(*([cbt] * _L))
    w2 = jax.scipy.linalg.block_diag(*([codebook] * _L))
    e = jax.scipy.linalg.block_diag(*([jnp.ones((1, _K), jnp.float32)] * _L))
    c2 = jnp.sum(codebook ** 2, axis=-1)
    c2seg = jnp.tile(c2, _L)[None, :]
    x2 = jnp.sum(latent.reshape(_B * _L, _D) ** 2, axis=-1).reshape(_B, _L)
    grid = _B // _BB
    q, pol, cbset_t = pl.pallas_call(
        _body,
        grid=(grid,),
        in_specs=[
            pl.BlockSpec((_BB, _L * _D), lambda i: (i, 0)),
            pl.BlockSpec((_BB, _L), lambda i: (i, 0)),
            pl.BlockSpec((1, _W), lambda i: (0, 0)),
            pl.BlockSpec((_L * _D, _W), lambda i: (0, 0)),
            pl.BlockSpec((_W, _L * _D), lambda i: (0, 0)),
            pl.BlockSpec((_L, _W), lambda i: (0, 0)),
            pl.BlockSpec((_D, _K), lambda i: (0, 0)),
        ],
        out_specs=[
            pl.BlockSpec((_BB, _L * _D), lambda i: (i, 0)),
            pl.BlockSpec((_BB, _L * _D), lambda i: (i, 0)),
            pl.BlockSpec((_BB, _D, _K), lambda i: (i, 0, 0)),
        ],
        out_shape=[
            jax.ShapeDtypeStruct((_B, _L * _D), jnp.float32),
            jax.ShapeDtypeStruct((_B, _L * _D), jnp.float32),
            jax.ShapeDtypeStruct((_B, _D, _K), jnp.float32),
        ],
    )(latent, x2, c2seg, w1, w2, e, cbt)
    return (pol, q, jnp.transpose(cbset_t, (0, 2, 1)))
